# Initial kernel scaffold; baseline (speedup 1.0000x reference)
import functools

import jax
import jax.numpy as jnp
from jax import lax
from jax.experimental import pallas as pl
from jax.experimental.pallas import tpu as pltpu
from jax.experimental.pallas import tpu_sc as plsc

_T, _B, _C = 1024, 128, 192
_REMAIN = _T - (_T * 3) // 4
_ROWS = _REMAIN * _B
_NC, _NS = 2, 16
_NW = _NC * _NS
_ROWS_PER_W = _ROWS // _NW
_CHUNK = 128
_NCHUNK = _ROWS_PER_W // _CHUNK
_L = 16


def _body(fwd_hbm, table_hbm, out_hbm, idx_hbm, raw_v, g_v, rows_v, sem):
    wid = lax.axis_index("s") * _NC + lax.axis_index("c")
    base_w = wid * _ROWS_PER_W
    lanes = lax.iota(jnp.int32, 16)
    for j in range(_NCHUNK):
        base = base_w + j * _CHUNK
        pltpu.sync_copy(fwd_hbm.at[pl.ds(base, _CHUNK)], raw_v)
        pltpu.sync_copy(raw_v, idx_hbm.at[pl.ds(base, _CHUNK)])
        for i in range(_CHUNK // _L):
            g_v[pl.ds(i * _L, _L)] = (
                raw_v[pl.ds(i * _L, _L)] * _B + lanes + (i * _L) % _B
            )
        pltpu.async_copy(table_hbm.at[g_v], rows_v, sem).wait()
        pltpu.sync_copy(rows_v, out_hbm.at[pl.ds(base, _CHUNK)])


@jax.jit
def _shuffle(fwd_flat, table):
    mesh = plsc.VectorSubcoreMesh(core_axis_name="c", subcore_axis_name="s")
    out, idx = pl.kernel(
        _body,
        out_type=(
            jax.ShapeDtypeStruct((_ROWS, _C), jnp.float32),
            jax.ShapeDtypeStruct((_ROWS,), jnp.int32),
        ),
        mesh=mesh,
        scratch_types=[
            pltpu.VMEM((_CHUNK,), jnp.int32),
            pltpu.VMEM((_CHUNK,), jnp.int32),
            pltpu.VMEM((_CHUNK, _C), jnp.float32),
            pltpu.SemaphoreType.DMA,
        ],
    )(fwd_flat, table)
    return out, idx


def kernel(patches, forward_indexes):
    table = patches.reshape(_T * _B, _C)
    fwd_flat = forward_indexes.reshape(_T * _B)
    out, idx = _shuffle(fwd_flat, table)
    return out.reshape(_REMAIN, _B, _C), idx.reshape(_REMAIN, _B)


# speedup vs baseline: 4.0995x; 4.0995x over previous
import functools

import jax
import jax.numpy as jnp
from jax import lax
from jax.experimental import pallas as pl
from jax.experimental.pallas import tpu as pltpu
from jax.experimental.pallas import tpu_sc as plsc

_T, _B, _C = 1024, 128, 192
_REMAIN = _T - (_T * 3) // 4
_ROWS = _REMAIN * _B
_NC, _NS = 2, 16
_NW = _NC * _NS
_ROWS_PER_W = _ROWS // _NW
_CHUNK = 128
_NCHUNK = _ROWS_PER_W // _CHUNK
_L = 16


def _body(fwd_hbm, table_hbm, out_hbm, idx_hbm, raw_v, g_v, rows_v, sem):
    wid = lax.axis_index("s") * _NC + lax.axis_index("c")
    base_w = wid * _ROWS_PER_W
    lanes = lax.iota(jnp.int32, 16)
    for j in range(_NCHUNK):
        base = base_w + j * _CHUNK
        pltpu.sync_copy(fwd_hbm.at[pl.ds(base, _CHUNK)], raw_v)
        pltpu.sync_copy(raw_v, idx_hbm.at[pl.ds(base, _CHUNK)])
        for i in range(_CHUNK // _L):
            g_v[pl.ds(i * _L, _L)] = (
                raw_v[pl.ds(i * _L, _L)] * _B + lanes + (i * _L) % _B
            )
        pltpu.async_copy(table_hbm.at[g_v], rows_v, sem).wait()
        pltpu.sync_copy(rows_v, out_hbm.at[pl.ds(base, _CHUNK)])


@jax.jit
def _shuffle(fwd_flat, table):
    mesh = plsc.VectorSubcoreMesh(core_axis_name="c", subcore_axis_name="s")
    out, idx = pl.kernel(
        _body,
        out_type=(
            jax.ShapeDtypeStruct((_ROWS, _C), jnp.float32),
            jax.ShapeDtypeStruct((_ROWS,), jnp.int32),
        ),
        mesh=mesh,
        compiler_params=pltpu.CompilerParams(use_tc_tiling_on_sc=False),
        scratch_types=[
            pltpu.VMEM((_CHUNK,), jnp.int32),
            pltpu.VMEM((_CHUNK,), jnp.int32),
            pltpu.VMEM((_CHUNK, _C), jnp.float32),
            pltpu.SemaphoreType.DMA,
        ],
    )(fwd_flat, table)
    return out, idx


def kernel(patches, forward_indexes):
    table = patches.reshape(_T * _B, _C)
    fwd_flat = forward_indexes.reshape(_T * _B)
    out, idx = _shuffle(fwd_flat, table)
    return out.reshape(_REMAIN, _B, _C), idx.reshape(_REMAIN, _B)


# speedup vs baseline: 4.2175x; 1.0288x over previous
import jax
import jax.numpy as jnp
from jax import lax
from jax.experimental import pallas as pl
from jax.experimental.pallas import tpu as pltpu
from jax.experimental.pallas import tpu_sc as plsc

_T, _B, _C = 1024, 128, 192
_REMAIN = _T - (_T * 3) // 4
_ROWS = _REMAIN * _B
_NC, _NS = 2, 16
_NW = _NC * _NS
_ROWS_PER_W = _ROWS // _NW
_CHUNK = 128
_NCHUNK = _ROWS_PER_W // _CHUNK
_NBUF = 4
_L = 16


def _body(fwd_hbm, table_hbm, out_hbm, idx_hbm, raw_v, g_v, rows, sem_i,
          sem_g, sem_s):
    wid = lax.axis_index("s") * _NC + lax.axis_index("c")
    base_w = wid * _ROWS_PER_W
    lanes = lax.iota(jnp.int32, 16)

    pltpu.sync_copy(fwd_hbm.at[pl.ds(base_w, _ROWS_PER_W)], raw_v)
    idx_cp = pltpu.async_copy(raw_v, idx_hbm.at[pl.ds(base_w, _ROWS_PER_W)],
                              sem_i)
    for j in range(_NCHUNK):
        for i in range(_CHUNK // _L):
            g_v[j, pl.ds(i * _L, _L)] = (
                raw_v[pl.ds(j * _CHUNK + i * _L, _L)] * _B
                + lanes + (i * _L) % _B
            )

    def start_gather(j):
        return pltpu.async_copy(table_hbm.at[g_v.at[j]], rows[j % _NBUF],
                                sem_g[j % _NBUF])

    gathers = {j: start_gather(j) for j in range(_NBUF)}
    stores = {}
    for j in range(_NCHUNK):
        if j >= 1 and j + _NBUF - 1 < _NCHUNK:
            stores[j - 1].wait()
            gathers[j + _NBUF - 1] = start_gather(j + _NBUF - 1)
        gathers[j].wait()
        stores[j] = pltpu.async_copy(
            rows[j % _NBUF],
            out_hbm.at[pl.ds(base_w + j * _CHUNK, _CHUNK)],
            sem_s[j % _NBUF])
    for j in range(_NCHUNK - min(_NBUF, _NCHUNK), _NCHUNK):
        stores[j].wait()
    idx_cp.wait()


@jax.jit
def _shuffle(fwd_flat, table):
    mesh = plsc.VectorSubcoreMesh(core_axis_name="c", subcore_axis_name="s")
    out, idx = pl.kernel(
        _body,
        out_type=(
            jax.ShapeDtypeStruct((_ROWS, _C), jnp.float32),
            jax.ShapeDtypeStruct((_ROWS,), jnp.int32),
        ),
        mesh=mesh,
        compiler_params=pltpu.CompilerParams(use_tc_tiling_on_sc=False),
        scratch_types=[
            pltpu.VMEM((_ROWS_PER_W,), jnp.int32),
            pltpu.VMEM((_NCHUNK, _CHUNK), jnp.int32),
            [pltpu.VMEM((_CHUNK, _C), jnp.float32) for _ in range(_NBUF)],
            pltpu.SemaphoreType.DMA,
            [pltpu.SemaphoreType.DMA for _ in range(_NBUF)],
            [pltpu.SemaphoreType.DMA for _ in range(_NBUF)],
        ],
    )(fwd_flat, table)
    return out, idx


def kernel(patches, forward_indexes):
    table = patches.reshape(_T * _B, _C)
    fwd_flat = forward_indexes.reshape(_T * _B)
    out, idx = _shuffle(fwd_flat, table)
    return out.reshape(_REMAIN, _B, _C), idx.reshape(_REMAIN, _B)
